# Initial kernel scaffold; baseline (speedup 1.0000x reference)
import jax
import jax.numpy as jnp
from jax.experimental import pallas as pl


def kernel(x, edge_index, W1, a_src1, a_dst1, b1, W2, a_src2, a_dst2, b2, W3, a_src3, a_dst3, b3):
    raise NotImplementedError("write your pallas kernel here")


# speedup vs baseline: 12.1718x; 12.1718x over previous
import functools

import jax
import jax.numpy as jnp
from jax import lax
from jax.experimental import pallas as pl
from jax.experimental.pallas import tpu as pltpu
from jax.experimental.pallas import tpu_sc as plsc

N = 10000
E = 320000
HEADS = 8
EN = E + N
NTEC = 16
CH_REAL = EN // NTEC
CH = 20640
PADDST = 1 << 30


def _tc_body(x_ref, w_ref, as_w_ref, ad_w_ref, h_ref, at_ref, k_ref):
    i = pl.program_id(0)
    h = jnp.dot(x_ref[...], w_ref[...], preferred_element_type=jnp.float32)
    h_ref[...] = h
    a_s = jnp.dot(h, as_w_ref[...], preferred_element_type=jnp.float32)
    a_d = jnp.dot(h, ad_w_ref[...], preferred_element_type=jnp.float32)
    at_ref[...] = jnp.concatenate([a_s, a_d], axis=1)
    kmax = jnp.stack([jnp.max(a_s, axis=0), jnp.max(a_d, axis=0)])

    @pl.when(i == 0)
    def _():
        k_ref[...] = kmax

    @pl.when(i != 0)
    def _():
        k_ref[...] = jnp.maximum(k_ref[...], kmax)


def _tc_layer(x, W, a_s, a_d):
    n, IN = x.shape
    HC = W.shape[1]
    C = HC // HEADS
    m = jnp.zeros((HEADS, C, 16), jnp.float32)
    As = m.at[jnp.arange(HEADS), :, jnp.arange(HEADS)].set(a_s).reshape(HC, 16)
    Ad = m.at[jnp.arange(HEADS), :, jnp.arange(HEADS)].set(a_d).reshape(HC, 16)
    bn = 1000
    grid = (n // bn,)
    return pl.pallas_call(
        _tc_body,
        grid=grid,
        in_specs=[
            pl.BlockSpec((bn, IN), lambda i: (i, 0)),
            pl.BlockSpec((IN, HC), lambda i: (0, 0)),
            pl.BlockSpec((HC, 16), lambda i: (0, 0)),
            pl.BlockSpec((HC, 16), lambda i: (0, 0)),
        ],
        out_specs=[
            pl.BlockSpec((bn, HC), lambda i: (i, 0)),
            pl.BlockSpec((bn, 32), lambda i: (i, 0)),
            pl.BlockSpec((2, 16), lambda i: (0, 0)),
        ],
        out_shape=[
            jax.ShapeDtypeStruct((n, HC), jnp.float32),
            jax.ShapeDtypeStruct((n, 32), jnp.float32),
            jax.ShapeDtypeStruct((2, 16), jnp.float32),
        ],
    )(x, W, As, Ad)


def _sc_body(C, S, slabs_per_sc, out_dim, is_final,
             src_hbm, dst_hbm, h_hbm, at_hbm, adt_hbm, k_hbm, b_hbm, out_hbm,
             src_chunk, dst_chunk, flt_idx, rows, at_rows, ad_slab,
             w_buf, gidx, sidx_buf, row_buf, out_buf, s_buf, k_buf, b_buf,
             sem1, sem2, acc_slab, s_slab):
    HC = HEADS * C
    cid = lax.axis_index("c")
    sid = lax.axis_index("s")
    iota = lax.iota(jnp.int32, 16)

    pltpu.sync_copy(src_hbm.at[sid], src_chunk)
    pltpu.sync_copy(dst_hbm.at[sid], dst_chunk)
    pltpu.sync_copy(k_hbm, k_buf)
    pltpu.sync_copy(b_hbm, b_buf)
    kvec = jnp.maximum(k_buf[0, :] + k_buf[1, :], 0.0)
    kv = [kvec[h] for h in range(HEADS)]

    for r in range(16):
        w_buf[r, :] = jnp.zeros((16,), jnp.float32)
    ad_slab[S, :] = jnp.zeros((16,), jnp.float32)

    def slab_loop(slab_i, _):
        slab = cid * slabs_per_sc + slab_i
        lo = slab * S
        r0 = sid * S // NTEC
        r1 = (sid + 1) * S // NTEC

        def _zrow(j, _):
            row_buf[pl.ds(j * 16, 16)] = jnp.zeros((16,), jnp.float32)
            return 0
        lax.fori_loop(0, HC // 16, _zrow, 0)

        def _zacc(r, _):
            pltpu.sync_copy(row_buf, acc_slab.at[r])
            pltpu.sync_copy(row_buf.at[pl.ds(0, 16)], s_slab.at[r])
            return 0
        lax.fori_loop(r0, r1, _zacc, 0)
        pltpu.sync_copy(adt_hbm.at[pl.ds(lo, S)], ad_slab.at[pl.ds(0, S)])
        plsc.subcore_barrier()

        def _scan(g, cursor):
            d = dst_chunk[pl.ds(g * 16, 16)]
            msk = (d >= lo) & (d < lo + S)
            cs = jnp.where(msk, 1, 0)
            for kk in (1, 2, 4, 8):
                idx = jnp.maximum(iota - kk, 0)
                sh = cs.at[idx].get(mode="promise_in_bounds")
                cs = cs + jnp.where(iota >= kk, sh, 0)
            pos = jnp.where(msk, cursor + cs - 1, CH + 8)
            plsc.store_scatter(flt_idx, [pos], iota + g * 16)
            return cursor + cs[15]
        cursor = lax.fori_loop(0, CH // 16, _scan, jnp.int32(0))

        rem = cursor % 16
        base = cursor - rem
        tail = flt_idx[pl.ds(base, 16)]
        flt_idx[pl.ds(base, 16)] = jnp.where(iota < rem, tail, CH - 1)
        nb = (cursor + 15) // 16

        def _batch(b, _):
            p = flt_idx[pl.ds(b * 16, 16)]
            sidx = plsc.load_gather(src_chunk, [p])
            dg = plsc.load_gather(dst_chunk, [p])
            didx = jnp.minimum(dg - lo, S)
            sidx_buf[...] = sidx
            cp1 = pltpu.async_copy(h_hbm.at[sidx_buf], rows, sem1)
            cp2 = pltpu.async_copy(at_hbm.at[sidx_buf], at_rows, sem2)
            cp1.wait()
            cp2.wait()
            for h in range(HEADS):
                a = plsc.load_gather(at_rows, [iota, jnp.full((16,), h, jnp.int32)])
                d = plsc.load_gather(ad_slab, [didx, jnp.full((16,), h, jnp.int32)])
                t = a + d
                e = jnp.where(t > 0, t, 0.2 * t)
                w = jnp.exp(e - kv[h])
                plsc.store_scatter(w_buf, [iota, jnp.full((16,), h, jnp.int32)], w)

            def _scale(i, _):
                wrow = w_buf[i, :]
                for h in range(HEADS):
                    wv = wrow[h]
                    for j in range(C // 16):
                        off = h * C + j * 16
                        rows[i, pl.ds(off, 16)] = rows[i, pl.ds(off, 16)] * wv
                return 0
            lax.fori_loop(0, 16, _scale, 0)
            gidx[...] = didx
            pltpu.sync_copy(rows, acc_slab.at[gidx], add=True)
            pltpu.sync_copy(w_buf, s_slab.at[gidx], add=True)
            return 0
        lax.fori_loop(0, nb, _batch, 0)
        plsc.subcore_barrier()

        def _flush(r, _):
            pltpu.sync_copy(acc_slab.at[r], row_buf)
            pltpu.sync_copy(s_slab.at[r], s_buf)
            svec = s_buf[...]
            if is_final:
                invv = 1.0 / (8.0 * (svec + 1e-16))
                inv = [invv[h] for h in range(HEADS)]
                for j in range(C // 16):
                    v = b_buf[pl.ds(j * 16, 16)]
                    for h in range(HEADS):
                        v = v + row_buf[pl.ds(h * C + j * 16, 16)] * inv[h]
                    out_buf[pl.ds(j * 16, 16)] = v
            else:
                invv = 1.0 / (svec + 1e-16)
                inv = [invv[h] for h in range(HEADS)]
                for h in range(HEADS):
                    for j in range(C // 16):
                        off = h * C + j * 16
                        v = row_buf[pl.ds(off, 16)] * inv[h] + b_buf[pl.ds(off, 16)]
                        v = jnp.where(v > 0, v, jnp.exp(v) - 1.0)
                        out_buf[pl.ds(off, 16)] = v
            pltpu.sync_copy(out_buf, out_hbm.at[lo + r])
            return 0
        lax.fori_loop(r0, r1, _flush, 0)
        plsc.subcore_barrier()
        return 0

    lax.fori_loop(0, slabs_per_sc, slab_loop, 0)


def _sc_layer(src2, dst2, h, at, k, b, C, S, is_final):
    HC = HEADS * C
    slabs_per_sc = N // S // 2
    out_dim = C if is_final else HC
    adt = at[:, 16:]
    body = functools.partial(_sc_body, C, S, slabs_per_sc, out_dim, is_final)
    kern = pl.kernel(
        body,
        out_type=jax.ShapeDtypeStruct((N, out_dim), jnp.float32),
        mesh=plsc.VectorSubcoreMesh(core_axis_name="c", subcore_axis_name="s"),
        compiler_params=pltpu.CompilerParams(use_tc_tiling_on_sc=False, needs_layout_passes=False),
        scratch_types=[
            pltpu.VMEM((CH,), jnp.int32),
            pltpu.VMEM((CH,), jnp.int32),
            pltpu.VMEM((CH + 16,), jnp.int32),
            pltpu.VMEM((16, HC), jnp.float32),
            pltpu.VMEM((16, 32), jnp.float32),
            pltpu.VMEM((S + 1, 16), jnp.float32),
            pltpu.VMEM((16, 16), jnp.float32),
            pltpu.VMEM((16,), jnp.int32),
            pltpu.VMEM((16,), jnp.int32),
            pltpu.VMEM((HC,), jnp.float32),
            pltpu.VMEM((out_dim,), jnp.float32),
            pltpu.VMEM((16,), jnp.float32),
            pltpu.VMEM((2, 16), jnp.float32),
            pltpu.VMEM((out_dim,), jnp.float32),
            pltpu.SemaphoreType.DMA,
            pltpu.SemaphoreType.DMA,
            pltpu.VMEM_SHARED((S + 1, HC), jnp.float32),
            pltpu.VMEM_SHARED((S + 1, 16), jnp.float32),
        ],
    )
    return kern(src2, dst2, h, at, adt, k, b)


def kernel(x, edge_index, W1, a_src1, a_dst1, b1, W2, a_src2, a_dst2, b2,
           W3, a_src3, a_dst3, b3):
    loop = jnp.arange(N, dtype=edge_index.dtype)
    src = jnp.concatenate([edge_index[0], loop]).astype(jnp.int32)
    dst = jnp.concatenate([edge_index[1], loop]).astype(jnp.int32)
    pad = CH - CH_REAL
    src2 = jnp.pad(src.reshape(NTEC, CH_REAL), ((0, 0), (0, pad)))
    dst2 = jnp.pad(dst.reshape(NTEC, CH_REAL), ((0, 0), (0, pad)),
                   constant_values=PADDST)

    h, at, k = _tc_layer(x, W1, a_src1, a_dst1)
    x = _sc_layer(src2, dst2, h, at, k, b1, 64, 625, False)
    h, at, k = _tc_layer(x, W2, a_src2, a_dst2)
    x = _sc_layer(src2, dst2, h, at, k, b2, 128, 500, False)
    h, at, k = _tc_layer(x, W3, a_src3, a_dst3)
    return _sc_layer(src2, dst2, h, at, k, b3, 256, 200, True)


# speedup vs baseline: 12.9294x; 1.0622x over previous
import functools

import jax
import jax.numpy as jnp
from jax import lax
from jax.experimental import pallas as pl
from jax.experimental.pallas import tpu as pltpu
from jax.experimental.pallas import tpu_sc as plsc

N = 10000
E = 320000
HEADS = 8
EN = E + N
NTEC = 16
CH_REAL = EN // NTEC
CH = 20640
PADDST = 1 << 30


def _tc_body(x_ref, w_ref, as_w_ref, ad_w_ref, hx_ref, ad_ref, k_ref):
    i = pl.program_id(0)
    h = jnp.dot(x_ref[...], w_ref[...], preferred_element_type=jnp.float32)
    a_s = jnp.dot(h, as_w_ref[...], preferred_element_type=jnp.float32)
    a_d = jnp.dot(h, ad_w_ref[...], preferred_element_type=jnp.float32)
    hx_ref[...] = jnp.concatenate([h, a_s], axis=1)
    ad_ref[...] = a_d
    kmax = jnp.stack([jnp.max(a_s, axis=0), jnp.max(a_d, axis=0)])

    @pl.when(i == 0)
    def _():
        k_ref[...] = kmax

    @pl.when(i != 0)
    def _():
        k_ref[...] = jnp.maximum(k_ref[...], kmax)


def _tc_layer(x, W, a_s, a_d):
    n, IN = x.shape
    HC = W.shape[1]
    C = HC // HEADS
    m = jnp.zeros((HEADS, C, 16), jnp.float32)
    As = m.at[jnp.arange(HEADS), :, jnp.arange(HEADS)].set(a_s).reshape(HC, 16)
    Ad = m.at[jnp.arange(HEADS), :, jnp.arange(HEADS)].set(a_d).reshape(HC, 16)
    bn = 1000
    grid = (n // bn,)
    return pl.pallas_call(
        _tc_body,
        grid=grid,
        in_specs=[
            pl.BlockSpec((bn, IN), lambda i: (i, 0)),
            pl.BlockSpec((IN, HC), lambda i: (0, 0)),
            pl.BlockSpec((HC, 16), lambda i: (0, 0)),
            pl.BlockSpec((HC, 16), lambda i: (0, 0)),
        ],
        out_specs=[
            pl.BlockSpec((bn, HC + 16), lambda i: (i, 0)),
            pl.BlockSpec((bn, 16), lambda i: (i, 0)),
            pl.BlockSpec((2, 16), lambda i: (0, 0)),
        ],
        out_shape=[
            jax.ShapeDtypeStruct((n, HC + 16), jnp.float32),
            jax.ShapeDtypeStruct((n, 16), jnp.float32),
            jax.ShapeDtypeStruct((2, 16), jnp.float32),
        ],
    )(x, W, As, Ad)


def _sc_body(C, S, slabs_per_sc, out_dim, is_final, pipelined,
             src_hbm, dst_hbm, hx_hbm, adt_hbm, k_hbm, b_hbm, out_hbm,
             src_chunk, dst_chunk, flt_idx, rows0, rows1, ad_slab,
             w_buf, didx0, didx1, sidx0, sidx1, row_buf, out_buf, s_buf,
             k_buf, b_buf, sem1, sem2, acc_slab, s_slab):
    HC = HEADS * C
    HW = HC + 16
    cid = lax.axis_index("c")
    sid = lax.axis_index("s")
    iota = lax.iota(jnp.int32, 16)

    pltpu.sync_copy(src_hbm.at[sid], src_chunk)
    pltpu.sync_copy(dst_hbm.at[sid], dst_chunk)
    pltpu.sync_copy(k_hbm, k_buf)
    pltpu.sync_copy(b_hbm, b_buf)
    kvec = jnp.maximum(k_buf[0, :] + k_buf[1, :], 0.0)
    kv = [kvec[h] for h in range(HEADS)]

    for r in range(16):
        w_buf[r, :] = jnp.zeros((16,), jnp.float32)
    ad_slab[S, :] = jnp.zeros((16,), jnp.float32)

    def _heads_scale_scatter(rows_b, didx_b, lo):
        dv = didx_b[...]
        for h in range(HEADS):
            a = plsc.load_gather(rows_b, [iota, jnp.full((16,), HC + h, jnp.int32)])
            d = plsc.load_gather(ad_slab, [dv, jnp.full((16,), h, jnp.int32)])
            t = a + d
            e = jnp.where(t > 0, t, 0.2 * t)
            w = jnp.exp(e - kv[h])
            plsc.store_scatter(w_buf, [iota, jnp.full((16,), h, jnp.int32)], w)

        def _scale(i, _):
            wrow = w_buf[i, :]
            for h in range(HEADS):
                wv = wrow[h]
                for j in range(C // 16):
                    off = h * C + j * 16
                    rows_b[i, pl.ds(off, 16)] = rows_b[i, pl.ds(off, 16)] * wv
            return 0
        lax.fori_loop(0, 16, _scale, 0)
        pltpu.sync_copy(rows_b, acc_slab.at[didx_b], add=True)
        pltpu.sync_copy(w_buf, s_slab.at[didx_b], add=True)

    def slab_loop(slab_i, _):
        slab = cid * slabs_per_sc + slab_i
        lo = slab * S
        r0 = sid * S // NTEC
        r1 = (sid + 1) * S // NTEC

        def _zrow(j, _):
            row_buf[pl.ds(j * 16, 16)] = jnp.zeros((16,), jnp.float32)
            return 0
        lax.fori_loop(0, HW // 16, _zrow, 0)

        def _zacc(r, _):
            pltpu.sync_copy(row_buf, acc_slab.at[r])
            pltpu.sync_copy(row_buf.at[pl.ds(0, 16)], s_slab.at[r])
            return 0
        lax.fori_loop(r0, r1, _zacc, 0)
        pltpu.sync_copy(adt_hbm.at[pl.ds(lo, S)], ad_slab.at[pl.ds(0, S)])
        plsc.subcore_barrier()

        def _scan(g, cursor):
            d = dst_chunk[pl.ds(g * 16, 16)]
            msk = (d >= lo) & (d < lo + S)
            cs = jnp.where(msk, 1, 0)
            for kk in (1, 2, 4, 8):
                idx = jnp.maximum(iota - kk, 0)
                sh = cs.at[idx].get(mode="promise_in_bounds")
                cs = cs + jnp.where(iota >= kk, sh, 0)
            pos = jnp.where(msk, cursor + cs - 1, CH + 40)
            plsc.store_scatter(flt_idx, [pos], iota + g * 16)
            return cursor + cs[15]
        cursor = lax.fori_loop(0, CH // 16, _scan, jnp.int32(0))

        rem = cursor % 16
        base = cursor - rem
        tail = flt_idx[pl.ds(base, 16)]
        flt_idx[pl.ds(base, 16)] = jnp.where(iota < rem, tail, CH - 1)
        flt_idx[pl.ds(base + 16, 16)] = jnp.full((16,), CH - 1, jnp.int32)
        flt_idx[pl.ds(base + 32, 16)] = jnp.full((16,), CH - 1, jnp.int32)

        def _idx_for(b, sidx_b, didx_b):
            p = flt_idx[pl.ds(b * 16, 16)]
            sidx_b[...] = plsc.load_gather(src_chunk, [p])
            dg = plsc.load_gather(dst_chunk, [p])
            didx_b[...] = jnp.minimum(dg - lo, S)

        if pipelined:
            npair = (cursor + 31) // 32
            _idx_for(0, sidx0, didx0)
            pltpu.async_copy(hx_hbm.at[sidx0], rows0, sem1).wait()

            def _pair(kp, _):
                _idx_for(2 * kp + 1, sidx1, didx1)
                cpb = pltpu.async_copy(hx_hbm.at[sidx1], rows1, sem2)
                _heads_scale_scatter(rows0, didx0, lo)
                cpb.wait()
                _idx_for(2 * kp + 2, sidx0, didx0)
                cpa = pltpu.async_copy(hx_hbm.at[sidx0], rows0, sem1)
                _heads_scale_scatter(rows1, didx1, lo)
                cpa.wait()
                return 0
            lax.fori_loop(0, npair, _pair, 0)
        else:
            nb = (cursor + 15) // 16

            def _batch(b, _):
                _idx_for(b, sidx0, didx0)
                pltpu.async_copy(hx_hbm.at[sidx0], rows0, sem1).wait()
                _heads_scale_scatter(rows0, didx0, lo)
                return 0
            lax.fori_loop(0, nb, _batch, 0)
        plsc.subcore_barrier()

        def _flush(r, _):
            pltpu.sync_copy(acc_slab.at[r], row_buf)
            pltpu.sync_copy(s_slab.at[r], s_buf)
            svec = s_buf[...]
            if is_final:
                invv = 1.0 / (8.0 * (svec + 1e-16))
                inv = [invv[h] for h in range(HEADS)]
                for j in range(C // 16):
                    v = b_buf[pl.ds(j * 16, 16)]
                    for h in range(HEADS):
                        v = v + row_buf[pl.ds(h * C + j * 16, 16)] * inv[h]
                    out_buf[pl.ds(j * 16, 16)] = v
            else:
                invv = 1.0 / (svec + 1e-16)
                inv = [invv[h] for h in range(HEADS)]
                for h in range(HEADS):
                    for j in range(C // 16):
                        off = h * C + j * 16
                        v = row_buf[pl.ds(off, 16)] * inv[h] + b_buf[pl.ds(off, 16)]
                        v = jnp.where(v > 0, v, jnp.exp(v) - 1.0)
                        out_buf[pl.ds(off, 16)] = v
            pltpu.sync_copy(out_buf, out_hbm.at[lo + r])
            return 0
        lax.fori_loop(r0, r1, _flush, 0)
        plsc.subcore_barrier()
        return 0

    lax.fori_loop(0, slabs_per_sc, slab_loop, 0)


def _sc_layer(src2, dst2, hx, adt, k, b, C, S, is_final, pipelined):
    HC = HEADS * C
    HW = HC + 16
    slabs_per_sc = N // S // 2
    out_dim = C if is_final else HC
    body = functools.partial(_sc_body, C, S, slabs_per_sc, out_dim, is_final,
                             pipelined)
    rows1_len = 16 if not pipelined else HW
    kern = pl.kernel(
        body,
        out_type=jax.ShapeDtypeStruct((N, out_dim), jnp.float32),
        mesh=plsc.VectorSubcoreMesh(core_axis_name="c", subcore_axis_name="s"),
        compiler_params=pltpu.CompilerParams(use_tc_tiling_on_sc=False, needs_layout_passes=False),
        scratch_types=[
            pltpu.VMEM((CH,), jnp.int32),
            pltpu.VMEM((CH,), jnp.int32),
            pltpu.VMEM((CH + 48,), jnp.int32),
            pltpu.VMEM((16, HW), jnp.float32),
            pltpu.VMEM((16, rows1_len), jnp.float32),
            pltpu.VMEM((S + 1, 16), jnp.float32),
            pltpu.VMEM((16, 16), jnp.float32),
            pltpu.VMEM((16,), jnp.int32),
            pltpu.VMEM((16,), jnp.int32),
            pltpu.VMEM((16,), jnp.int32),
            pltpu.VMEM((16,), jnp.int32),
            pltpu.VMEM((HW,), jnp.float32),
            pltpu.VMEM((out_dim,), jnp.float32),
            pltpu.VMEM((16,), jnp.float32),
            pltpu.VMEM((2, 16), jnp.float32),
            pltpu.VMEM((out_dim,), jnp.float32),
            pltpu.SemaphoreType.DMA,
            pltpu.SemaphoreType.DMA,
            pltpu.VMEM_SHARED((S + 1, HW), jnp.float32),
            pltpu.VMEM_SHARED((S + 1, 16), jnp.float32),
        ],
    )
    return kern(src2, dst2, hx, adt, k, b)


def kernel(x, edge_index, W1, a_src1, a_dst1, b1, W2, a_src2, a_dst2, b2,
           W3, a_src3, a_dst3, b3):
    loop = jnp.arange(N, dtype=edge_index.dtype)
    src = jnp.concatenate([edge_index[0], loop]).astype(jnp.int32)
    dst = jnp.concatenate([edge_index[1], loop]).astype(jnp.int32)
    pad = CH - CH_REAL
    src2 = jnp.pad(src.reshape(NTEC, CH_REAL), ((0, 0), (0, pad)))
    dst2 = jnp.pad(dst.reshape(NTEC, CH_REAL), ((0, 0), (0, pad)),
                   constant_values=PADDST)

    hx, adt, k = _tc_layer(x, W1, a_src1, a_dst1)
    x = _sc_layer(src2, dst2, hx, adt, k, b1, 64, 625, False, True)
    hx, adt, k = _tc_layer(x, W2, a_src2, a_dst2)
    x = _sc_layer(src2, dst2, hx, adt, k, b2, 128, 250, False, True)
    hx, adt, k = _tc_layer(x, W3, a_src3, a_dst3)
    return _sc_layer(src2, dst2, hx, adt, k, b3, 256, 200, True, False)


# speedup vs baseline: 13.7680x; 1.0649x over previous
import functools

import jax
import jax.numpy as jnp
from jax import lax
from jax.experimental import pallas as pl
from jax.experimental.pallas import tpu as pltpu
from jax.experimental.pallas import tpu_sc as plsc

N = 10000
E = 320000
HEADS = 8
EN = E + N
NTEC = 16
CH_REAL = EN // NTEC
CH = 20640
PADDST = 1 << 30


def _tc_body(x_ref, w_ref, as_w_ref, ad_w_ref, hx_ref, ad_ref, k_ref):
    i = pl.program_id(0)
    h = jnp.dot(x_ref[...], w_ref[...], preferred_element_type=jnp.float32)
    a_s = jnp.dot(h, as_w_ref[...], preferred_element_type=jnp.float32)
    a_d = jnp.dot(h, ad_w_ref[...], preferred_element_type=jnp.float32)
    hx_ref[...] = jnp.concatenate([h, a_s], axis=1)
    ad_ref[...] = a_d
    kmax = jnp.stack([jnp.max(a_s, axis=0), jnp.max(a_d, axis=0)])

    @pl.when(i == 0)
    def _():
        k_ref[...] = kmax

    @pl.when(i != 0)
    def _():
        k_ref[...] = jnp.maximum(k_ref[...], kmax)


def _tc_layer(x, W, a_s, a_d):
    n, IN = x.shape
    HC = W.shape[1]
    C = HC // HEADS
    m = jnp.zeros((HEADS, C, 16), jnp.float32)
    As = m.at[jnp.arange(HEADS), :, jnp.arange(HEADS)].set(a_s).reshape(HC, 16)
    Ad = m.at[jnp.arange(HEADS), :, jnp.arange(HEADS)].set(a_d).reshape(HC, 16)
    bn = 1000
    grid = (n // bn,)
    return pl.pallas_call(
        _tc_body,
        grid=grid,
        in_specs=[
            pl.BlockSpec((bn, IN), lambda i: (i, 0)),
            pl.BlockSpec((IN, HC), lambda i: (0, 0)),
            pl.BlockSpec((HC, 16), lambda i: (0, 0)),
            pl.BlockSpec((HC, 16), lambda i: (0, 0)),
        ],
        out_specs=[
            pl.BlockSpec((bn, HC + 16), lambda i: (i, 0)),
            pl.BlockSpec((bn, 16), lambda i: (i, 0)),
            pl.BlockSpec((2, 16), lambda i: (0, 0)),
        ],
        out_shape=[
            jax.ShapeDtypeStruct((n, HC + 16), jnp.float32),
            jax.ShapeDtypeStruct((n, 16), jnp.float32),
            jax.ShapeDtypeStruct((2, 16), jnp.float32),
        ],
    )(x, W, As, Ad)


def _sc_body(C, S, slabs_per_sc, out_dim, is_final, pipelined,
             src_hbm, dst_hbm, hx_hbm, adt_hbm, k_hbm, b_hbm, out_hbm,
             src_chunk, dst_chunk, flt_idx, rows0, rows1, ad_slab,
             w_buf, didx0, didx1, sidx0, sidx1, row_buf, out_buf, s_buf,
             k_buf, b_buf, sem1, sem2, acc_slab, s_slab):
    HC = HEADS * C
    HW = HC + 16
    cid = lax.axis_index("c")
    sid = lax.axis_index("s")
    iota = lax.iota(jnp.int32, 16)

    pltpu.sync_copy(src_hbm.at[sid], src_chunk)
    pltpu.sync_copy(dst_hbm.at[sid], dst_chunk)
    pltpu.sync_copy(k_hbm, k_buf)
    pltpu.sync_copy(b_hbm, b_buf)
    kvec = jnp.maximum(k_buf[0, :] + k_buf[1, :], 0.0)
    kv = [kvec[h] for h in range(HEADS)]

    for r in range(16):
        w_buf[r, :] = jnp.zeros((16,), jnp.float32)
    ad_slab[S, :] = jnp.zeros((16,), jnp.float32)

    def _heads_scale_scatter(rows_b, didx_b, lo):
        dv = didx_b[...]
        for h in range(HEADS):
            a = plsc.load_gather(rows_b, [iota, jnp.full((16,), HC + h, jnp.int32)])
            d = plsc.load_gather(ad_slab, [dv, jnp.full((16,), h, jnp.int32)])
            t = a + d
            e = jnp.where(t > 0, t, 0.2 * t)
            w = jnp.exp(e - kv[h])
            plsc.store_scatter(w_buf, [iota, jnp.full((16,), h, jnp.int32)], w)

        for i in range(16):
            wrow = w_buf[i, :]
            for h in range(HEADS):
                wv = wrow[h]
                for j in range(C // 16):
                    off = h * C + j * 16
                    rows_b[i, pl.ds(off, 16)] = rows_b[i, pl.ds(off, 16)] * wv
        pltpu.sync_copy(rows_b, acc_slab.at[didx_b], add=True)
        pltpu.sync_copy(w_buf, s_slab.at[didx_b], add=True)

    def slab_loop(slab_i, _):
        slab = cid * slabs_per_sc + slab_i
        lo = slab * S
        r0 = sid * S // NTEC
        r1 = (sid + 1) * S // NTEC

        def _zrow(j, _):
            row_buf[pl.ds(j * 16, 16)] = jnp.zeros((16,), jnp.float32)
            return 0
        lax.fori_loop(0, HW // 16, _zrow, 0)

        def _zacc(r, _):
            pltpu.sync_copy(row_buf, acc_slab.at[r])
            pltpu.sync_copy(row_buf.at[pl.ds(0, 16)], s_slab.at[r])
            return 0
        lax.fori_loop(r0, r1, _zacc, 0)
        pltpu.sync_copy(adt_hbm.at[pl.ds(lo, S)], ad_slab.at[pl.ds(0, S)])
        plsc.subcore_barrier()

        def _scan(g, cursor):
            d = dst_chunk[pl.ds(g * 16, 16)]
            msk = (d >= lo) & (d < lo + S)
            cs = jnp.where(msk, 1, 0)
            for kk in (1, 2, 4, 8):
                idx = jnp.maximum(iota - kk, 0)
                sh = cs.at[idx].get(mode="promise_in_bounds")
                cs = cs + jnp.where(iota >= kk, sh, 0)
            pos = jnp.where(msk, cursor + cs - 1, CH + 40)
            plsc.store_scatter(flt_idx, [pos], iota + g * 16)
            return cursor + cs[15]
        cursor = lax.fori_loop(0, CH // 16, _scan, jnp.int32(0))

        rem = cursor % 16
        base = cursor - rem
        tail = flt_idx[pl.ds(base, 16)]
        flt_idx[pl.ds(base, 16)] = jnp.where(iota < rem, tail, CH - 1)
        flt_idx[pl.ds(base + 16, 16)] = jnp.full((16,), CH - 1, jnp.int32)
        flt_idx[pl.ds(base + 32, 16)] = jnp.full((16,), CH - 1, jnp.int32)

        def _idx_for(b, sidx_b, didx_b):
            p = flt_idx[pl.ds(b * 16, 16)]
            sidx_b[...] = plsc.load_gather(src_chunk, [p])
            dg = plsc.load_gather(dst_chunk, [p])
            didx_b[...] = jnp.minimum(dg - lo, S)

        if pipelined:
            npair = (cursor + 31) // 32
            _idx_for(0, sidx0, didx0)
            pltpu.async_copy(hx_hbm.at[sidx0], rows0, sem1).wait()

            def _pair(kp, _):
                _idx_for(2 * kp + 1, sidx1, didx1)
                cpb = pltpu.async_copy(hx_hbm.at[sidx1], rows1, sem2)
                _heads_scale_scatter(rows0, didx0, lo)
                cpb.wait()
                _idx_for(2 * kp + 2, sidx0, didx0)
                cpa = pltpu.async_copy(hx_hbm.at[sidx0], rows0, sem1)
                _heads_scale_scatter(rows1, didx1, lo)
                cpa.wait()
                return 0
            lax.fori_loop(0, npair, _pair, 0)
        else:
            nb = (cursor + 15) // 16

            def _batch(b, _):
                _idx_for(b, sidx0, didx0)
                pltpu.async_copy(hx_hbm.at[sidx0], rows0, sem1).wait()
                _heads_scale_scatter(rows0, didx0, lo)
                return 0
            lax.fori_loop(0, nb, _batch, 0)
        plsc.subcore_barrier()

        def _flush(r, _):
            pltpu.sync_copy(acc_slab.at[r], row_buf)
            pltpu.sync_copy(s_slab.at[r], s_buf)
            svec = s_buf[...]
            if is_final:
                invv = 1.0 / (8.0 * (svec + 1e-16))
                inv = [invv[h] for h in range(HEADS)]
                for j in range(C // 16):
                    v = b_buf[pl.ds(j * 16, 16)]
                    for h in range(HEADS):
                        v = v + row_buf[pl.ds(h * C + j * 16, 16)] * inv[h]
                    out_buf[pl.ds(j * 16, 16)] = v
            else:
                invv = 1.0 / (svec + 1e-16)
                inv = [invv[h] for h in range(HEADS)]
                for h in range(HEADS):
                    for j in range(C // 16):
                        off = h * C + j * 16
                        v = row_buf[pl.ds(off, 16)] * inv[h] + b_buf[pl.ds(off, 16)]
                        v = jnp.where(v > 0, v, jnp.exp(v) - 1.0)
                        out_buf[pl.ds(off, 16)] = v
            pltpu.sync_copy(out_buf, out_hbm.at[lo + r])
            return 0
        lax.fori_loop(r0, r1, _flush, 0)
        plsc.subcore_barrier()
        return 0

    lax.fori_loop(0, slabs_per_sc, slab_loop, 0)


def _sc_layer(src2, dst2, hx, adt, k, b, C, S, is_final, pipelined):
    HC = HEADS * C
    HW = HC + 16
    slabs_per_sc = N // S // 2
    out_dim = C if is_final else HC
    body = functools.partial(_sc_body, C, S, slabs_per_sc, out_dim, is_final,
                             pipelined)
    rows1_len = 16 if not pipelined else HW
    kern = pl.kernel(
        body,
        out_type=jax.ShapeDtypeStruct((N, out_dim), jnp.float32),
        mesh=plsc.VectorSubcoreMesh(core_axis_name="c", subcore_axis_name="s"),
        compiler_params=pltpu.CompilerParams(use_tc_tiling_on_sc=False, needs_layout_passes=False),
        scratch_types=[
            pltpu.VMEM((CH,), jnp.int32),
            pltpu.VMEM((CH,), jnp.int32),
            pltpu.VMEM((CH + 48,), jnp.int32),
            pltpu.VMEM((16, HW), jnp.float32),
            pltpu.VMEM((16, rows1_len), jnp.float32),
            pltpu.VMEM((S + 1, 16), jnp.float32),
            pltpu.VMEM((16, 16), jnp.float32),
            pltpu.VMEM((16,), jnp.int32),
            pltpu.VMEM((16,), jnp.int32),
            pltpu.VMEM((16,), jnp.int32),
            pltpu.VMEM((16,), jnp.int32),
            pltpu.VMEM((HW,), jnp.float32),
            pltpu.VMEM((out_dim,), jnp.float32),
            pltpu.VMEM((16,), jnp.float32),
            pltpu.VMEM((2, 16), jnp.float32),
            pltpu.VMEM((out_dim,), jnp.float32),
            pltpu.SemaphoreType.DMA,
            pltpu.SemaphoreType.DMA,
            pltpu.VMEM_SHARED((S + 1, HW), jnp.float32),
            pltpu.VMEM_SHARED((S + 1, 16), jnp.float32),
        ],
    )
    return kern(src2, dst2, hx, adt, k, b)


def kernel(x, edge_index, W1, a_src1, a_dst1, b1, W2, a_src2, a_dst2, b2,
           W3, a_src3, a_dst3, b3):
    loop = jnp.arange(N, dtype=edge_index.dtype)
    src = jnp.concatenate([edge_index[0], loop]).astype(jnp.int32)
    dst = jnp.concatenate([edge_index[1], loop]).astype(jnp.int32)
    pad = CH - CH_REAL
    src2 = jnp.pad(src.reshape(NTEC, CH_REAL), ((0, 0), (0, pad)))
    dst2 = jnp.pad(dst.reshape(NTEC, CH_REAL), ((0, 0), (0, pad)),
                   constant_values=PADDST)

    hx, adt, k = _tc_layer(x, W1, a_src1, a_dst1)
    x = _sc_layer(src2, dst2, hx, adt, k, b1, 64, 625, False, True)
    hx, adt, k = _tc_layer(x, W2, a_src2, a_dst2)
    x = _sc_layer(src2, dst2, hx, adt, k, b2, 128, 250, False, True)
    hx, adt, k = _tc_layer(x, W3, a_src3, a_dst3)
    return _sc_layer(src2, dst2, hx, adt, k, b3, 256, 200, True, False)
